# Initial kernel scaffold; baseline (speedup 1.0000x reference)
import jax
import jax.numpy as jnp
from jax.experimental import pallas as pl


def kernel(inputs, W1, b1, W2, b2, Wc, bc):
    raise NotImplementedError("write your pallas kernel here")


# speedup vs baseline: 5.1896x; 5.1896x over previous
import functools

import jax
import jax.numpy as jnp
from jax.experimental import pallas as pl

_B, _F, _P = 4, 8, 2048
_NT = _F - 1
_BT = _B * _NT
_RB = 256
_NRB = _P // _RB
_KS = (5, 15, 40)


def _acos_poly(x):
    p = jnp.float32(-0.0012624911)
    p = p * x + jnp.float32(0.0066700901)
    p = p * x + jnp.float32(-0.0170881256)
    p = p * x + jnp.float32(0.0308918810)
    p = p * x + jnp.float32(-0.0501743046)
    p = p * x + jnp.float32(0.0889789874)
    p = p * x + jnp.float32(-0.2145988016)
    p = p * x + jnp.float32(1.5707963050)
    return jnp.sqrt(jnp.maximum(1.0 - x, 0.0)) * p


def _erf_poly(x):
    ax = jnp.abs(x)
    t = 1.0 / (1.0 + jnp.float32(0.3275911) * ax)
    y = jnp.float32(1.061405429)
    y = y * t + jnp.float32(-1.453152027)
    y = y * t + jnp.float32(1.421413741)
    y = y * t + jnp.float32(-0.284496736)
    y = y * t + jnp.float32(0.254829592)
    y = 1.0 - y * t * jnp.exp(-ax * ax)
    return jnp.where(x >= 0, y, -y)


def _gelu(x):
    return x * 0.5 * (1.0 + _erf_poly(x * jnp.float32(0.7071067811865476)))


def _bearing(x, y, z):
    cx = (jnp.min(x, axis=1, keepdims=True) + jnp.max(x, axis=1, keepdims=True)) * 0.5
    cy = (jnp.min(y, axis=1, keepdims=True) + jnp.max(y, axis=1, keepdims=True)) * 0.5
    cz = (jnp.min(z, axis=1, keepdims=True) + jnp.max(z, axis=1, keepdims=True)) * 0.5
    dx, dy, dz = x - cx, y - cy, z - cz
    n = jnp.maximum(jnp.sqrt(dx * dx + dy * dy + dz * dz), jnp.float32(1e-12))
    dx, dy, dz = dx / n, dy / n, dz / n
    dot = jnp.clip(dy, -1.0 + 1e-7, 1.0 - 1e-7)
    w = jnp.sqrt((1.0 + dot) * 0.5)
    s = jnp.sqrt((1.0 - dot) * 0.5)
    an = jnp.maximum(jnp.sqrt(dx * dx + dz * dz), jnp.float32(1e-12))
    return w, dz / an * s, -dx / an * s


def _k1_body(xa, ya, za, xb, yb, zb, qw_o, qx_o, qy_o, qz_o):
    bw, bx, bz = _bearing(xa[0], ya[0], za[0])
    aw, ax, az = _bearing(xb[0], yb[0], zb[0])
    qw = aw * bw + ax * bx + az * bz
    qx = ax * bw - aw * bx
    qy = ax * bz - az * bx
    qz = az * bw - aw * bz
    n = jnp.maximum(jnp.sqrt(qw * qw + qx * qx + qy * qy + qz * qz),
                    jnp.float32(1e-12))
    qw_o[0] = qw / n
    qx_o[0] = qx / n
    qy_o[0] = qy / n
    qz_o[0] = qz / n


def _quat_prep(xa, ya, za, xb, yb, zb):
    spec = pl.BlockSpec((1, _NT, _P), lambda b: (b, 0, 0))
    shp = jax.ShapeDtypeStruct((_B, _NT, _P), jnp.float32)
    return pl.pallas_call(
        _k1_body,
        grid=(_B,),
        in_specs=[spec] * 6,
        out_specs=[spec] * 4,
        out_shape=[shp] * 4,
    )(xa, ya, za, xb, yb, zb)


def _k2_body(pmT_ref, pm_ref, qT_ref, q_ref, o5, o15, o40):
    pmT = pmT_ref[0]
    pm = pm_ref[0]
    qT = qT_ref[0]
    q = q_ref[0]
    g = jnp.dot(pm, pmT, preferred_element_type=jnp.float32)
    sqr = jnp.sum(pm * pm, axis=1, keepdims=True)
    sqc = jnp.sum(pmT * pmT, axis=0, keepdims=True)
    d = -((sqr + sqc) - 2.0 * g)
    dp = jnp.abs(jnp.dot(q, qT, preferred_element_type=jnp.float32,
                         precision=jax.lax.Precision.HIGHEST))
    jidx = jax.lax.broadcasted_iota(jnp.int32, (_RB, _P), 1)
    neg = jnp.float32(-3.0e38)

    def step(_, carry):
        dcur, acc = carry
        m = jnp.max(dcur, axis=1, keepdims=True)
        cand = jnp.where(dcur >= m, jidx, jnp.int32(_P))
        amin = jnp.min(cand, axis=1, keepdims=True)
        sel = jidx == amin
        v = jnp.sum(jnp.where(sel, dp, 0.0), axis=1)
        a = 2.0 * _acos_poly(jnp.clip(v, 0.0, 1.0 - 1e-7))
        return jnp.where(sel, neg, dcur), acc + a

    acc0 = jnp.zeros((_RB,), jnp.float32)
    dcur, acc = jax.lax.fori_loop(0, _KS[0], step, (d, acc0))
    o5[...] = (acc * jnp.float32(1.0 / _KS[0])).reshape(1, 1, 1, _RB)
    dcur, acc = jax.lax.fori_loop(_KS[0], _KS[1], step, (dcur, acc))
    o15[...] = (acc * jnp.float32(1.0 / _KS[1])).reshape(1, 1, 1, _RB)
    dcur, acc = jax.lax.fori_loop(_KS[1], _KS[2], step, (dcur, acc))
    o40[...] = (acc * jnp.float32(1.0 / _KS[2])).reshape(1, 1, 1, _RB)


def _knn_angles(pmT, pm, qT, q):
    full = pl.BlockSpec((1, 8, _P), lambda i, j: (i, 0, 0))
    rows = pl.BlockSpec((1, _RB, 8), lambda i, j: (i, j, 0))
    ospec = pl.BlockSpec((1, 1, 1, _RB), lambda i, j: (i, j, 0, 0))
    oshp = jax.ShapeDtypeStruct((_BT, _NRB, 1, _RB), jnp.float32)
    return pl.pallas_call(
        _k2_body,
        grid=(_BT, _NRB),
        in_specs=[full, rows, full, rows],
        out_specs=[ospec] * 3,
        out_shape=[oshp] * 3,
    )(pmT, pm, qT, q)


def _kth_bits(bits, kk):
    def body(_, lohi):
        lo, hi = lohi
        mid = lo + ((hi - lo) >> 1)
        cnt = jnp.sum((bits <= mid).astype(jnp.int32))
        pred = cnt >= kk
        return (jnp.where(pred, lo, mid + 1), jnp.where(pred, mid, hi))

    lo, _ = jax.lax.fori_loop(
        0, 31, body, (jnp.int32(0), jnp.int32(0x7F7FFFFF)))
    return lo


def _k3_body(i5, i15, i40, r5, r15, r40):
    half = _B * _P // 2
    for inc_ref, out_ref in ((i5, r5), (i15, r15), (i40, r40)):
        mi = jnp.mean(inc_ref[...], axis=1)
        bits = jax.lax.bitcast_convert_type(mi, jnp.int32)
        v1 = jax.lax.bitcast_convert_type(_kth_bits(bits, half), jnp.float32)
        v2 = jax.lax.bitcast_convert_type(_kth_bits(bits, half + 1),
                                          jnp.float32)
        scale = jnp.maximum((v1 + v2) * 0.5, jnp.float32(1e-6))
        out_ref[...] = jnp.exp(-mi / scale).reshape(_B, 1, _P)


def _rigidity(inc5, inc15, inc40):
    ispec = pl.BlockSpec((_B, _NT, _P), lambda: (0, 0, 0))
    ospec = pl.BlockSpec((_B, 1, _P), lambda: (0, 0, 0))
    oshp = jax.ShapeDtypeStruct((_B, 1, _P), jnp.float32)
    return pl.pallas_call(
        _k3_body,
        grid=(),
        in_specs=[ispec] * 3,
        out_specs=[ospec] * 3,
        out_shape=[oshp] * 3,
    )(inc5, inc15, inc40)


def _k4_body(p0, p1, p2, p3, r5, r15, r40, w1t, b1c, w2t, b2c, mx_o, sm_o):
    f = pl.program_id(1)
    zero = jnp.zeros((1, _P), jnp.float32)
    xr = jnp.concatenate(
        [p0[0], p1[0], p2[0], p3[0], r5[0], r15[0], r40[0], zero], axis=0)
    h1 = jnp.dot(w1t[...], xr, preferred_element_type=jnp.float32) + b1c[...]
    h1 = _gelu(h1)
    h2 = jnp.dot(w2t[...], h1, preferred_element_type=jnp.float32) + b2c[...]
    h2 = _gelu(h2)
    mx = jnp.max(h2, axis=1).reshape(1, 1, -1)
    sm = jnp.sum(h2, axis=1).reshape(1, 1, -1)

    @pl.when(f == 0)
    def _init():
        mx_o[...] = mx
        sm_o[...] = sm

    @pl.when(f > 0)
    def _acc():
        mx_o[...] = jnp.maximum(mx_o[...], mx)
        sm_o[...] = sm_o[...] + sm


def _mlp_pool(pts, rigs, w1t, b1c, w2t, b2c):
    h1, h2 = w1t.shape[0], w2t.shape[0]
    pspec = pl.BlockSpec((1, 1, _P), lambda b, f: (b * _F + f, 0, 0))
    rspec = pl.BlockSpec((1, 1, _P), lambda b, f: (b, 0, 0))
    w1spec = pl.BlockSpec(w1t.shape, lambda b, f: (0, 0))
    b1spec = pl.BlockSpec(b1c.shape, lambda b, f: (0, 0))
    w2spec = pl.BlockSpec(w2t.shape, lambda b, f: (0, 0))
    b2spec = pl.BlockSpec(b2c.shape, lambda b, f: (0, 0))
    ospec = pl.BlockSpec((1, 1, h2), lambda b, f: (b, 0, 0))
    oshp = jax.ShapeDtypeStruct((_B, 1, h2), jnp.float32)
    return pl.pallas_call(
        _k4_body,
        grid=(_B, _F),
        in_specs=[pspec] * 4 + [rspec] * 3 + [w1spec, b1spec, w2spec, b2spec],
        out_specs=[ospec] * 2,
        out_shape=[oshp] * 2,
    )(*pts, *rigs, w1t, b1c, w2t, b2c)


def _k5_body(mx, sm, wc, bc, out):
    pooled = jnp.concatenate(
        [mx[:, 0, :], sm[:, 0, :] * jnp.float32(1.0 / (_F * _P))], axis=1)
    out[...] = jnp.dot(pooled, wc[...],
                       preferred_element_type=jnp.float32) + bc[...]


def _head(mx, sm, wc, bcr):
    nc = wc.shape[1]
    return pl.pallas_call(
        _k5_body,
        grid=(),
        in_specs=[pl.BlockSpec(mx.shape, lambda: (0, 0, 0)),
                  pl.BlockSpec(sm.shape, lambda: (0, 0, 0)),
                  pl.BlockSpec(wc.shape, lambda: (0, 0)),
                  pl.BlockSpec(bcr.shape, lambda: (0, 0))],
        out_specs=pl.BlockSpec((_B, nc), lambda: (0, 0)),
        out_shape=jax.ShapeDtypeStruct((_B, nc), jnp.float32),
    )(mx, sm, wc, bcr)


def kernel(inputs, W1, b1, W2, b2, Wc, bc):
    f32 = jnp.float32
    x = inputs[..., 0].astype(f32)
    y = inputs[..., 1].astype(f32)
    z = inputs[..., 2].astype(f32)

    qw, qx, qy, qz = _quat_prep(x[:, :_NT], y[:, :_NT], z[:, :_NT],
                                x[:, 1:], y[:, 1:], z[:, 1:])

    zero = jnp.zeros((_B, _NT, _P), f32)
    pmT = jnp.stack([x[:, :_NT], y[:, :_NT], z[:, :_NT],
                     zero, zero, zero, zero, zero],
                    axis=2).reshape(_BT, 8, _P)
    qT = jnp.stack([qw, qx, qy, qz, zero, zero, zero, zero],
                   axis=2).reshape(_BT, 8, _P)
    pm = pmT.transpose(0, 2, 1)
    q = qT.transpose(0, 2, 1)

    inc5, inc15, inc40 = _knn_angles(pmT, pm, qT, q)
    inc5 = inc5.reshape(_B, _NT, _P)
    inc15 = inc15.reshape(_B, _NT, _P)
    inc40 = inc40.reshape(_B, _NT, _P)

    rigs = _rigidity(inc5, inc15, inc40)

    pts = [inputs[..., c].astype(f32).reshape(_B * _F, 1, _P)
           for c in range(4)]
    w1t = jnp.zeros((W1.shape[1], 8), f32).at[:, :7].set(W1.T.astype(f32))
    b1c = b1.astype(f32).reshape(-1, 1)
    w2t = W2.T.astype(f32)
    b2c = b2.astype(f32).reshape(-1, 1)
    mx, sm = _mlp_pool(pts, rigs, w1t, b1c, w2t, b2c)

    return _head(mx, sm, Wc.astype(f32), bc.astype(f32).reshape(1, -1))


# speedup vs baseline: 6.1570x; 1.1864x over previous
import functools

import jax
import jax.numpy as jnp
from jax.experimental import pallas as pl
from jax.experimental.pallas import tpu as pltpu

_B, _F, _P = 4, 8, 2048
_NT = _F - 1
_BT = _B * _NT
_RB = 256
_NRB = _P // _RB
_KS = (5, 15, 40)


def _acos_poly(x):
    p = jnp.float32(-0.0012624911)
    p = p * x + jnp.float32(0.0066700901)
    p = p * x + jnp.float32(-0.0170881256)
    p = p * x + jnp.float32(0.0308918810)
    p = p * x + jnp.float32(-0.0501743046)
    p = p * x + jnp.float32(0.0889789874)
    p = p * x + jnp.float32(-0.2145988016)
    p = p * x + jnp.float32(1.5707963050)
    return jnp.sqrt(jnp.maximum(1.0 - x, 0.0)) * p


def _erf_poly(x):
    ax = jnp.abs(x)
    t = 1.0 / (1.0 + jnp.float32(0.3275911) * ax)
    y = jnp.float32(1.061405429)
    y = y * t + jnp.float32(-1.453152027)
    y = y * t + jnp.float32(1.421413741)
    y = y * t + jnp.float32(-0.284496736)
    y = y * t + jnp.float32(0.254829592)
    y = 1.0 - y * t * jnp.exp(-ax * ax)
    return jnp.where(x >= 0, y, -y)


def _gelu(x):
    return x * 0.5 * (1.0 + _erf_poly(x * jnp.float32(0.7071067811865476)))


def _bearing(x, y, z):
    cx = (jnp.min(x, axis=1, keepdims=True) + jnp.max(x, axis=1, keepdims=True)) * 0.5
    cy = (jnp.min(y, axis=1, keepdims=True) + jnp.max(y, axis=1, keepdims=True)) * 0.5
    cz = (jnp.min(z, axis=1, keepdims=True) + jnp.max(z, axis=1, keepdims=True)) * 0.5
    dx, dy, dz = x - cx, y - cy, z - cz
    n = jnp.maximum(jnp.sqrt(dx * dx + dy * dy + dz * dz), jnp.float32(1e-12))
    dx, dy, dz = dx / n, dy / n, dz / n
    dot = jnp.clip(dy, -1.0 + 1e-7, 1.0 - 1e-7)
    w = jnp.sqrt((1.0 + dot) * 0.5)
    s = jnp.sqrt((1.0 - dot) * 0.5)
    an = jnp.maximum(jnp.sqrt(dx * dx + dz * dz), jnp.float32(1e-12))
    return w, dz / an * s, -dx / an * s


def _k1_body(xa, ya, za, xb, yb, zb, qw_o, qx_o, qy_o, qz_o):
    bw, bx, bz = _bearing(xa[0], ya[0], za[0])
    aw, ax, az = _bearing(xb[0], yb[0], zb[0])
    qw = aw * bw + ax * bx + az * bz
    qx = ax * bw - aw * bx
    qy = ax * bz - az * bx
    qz = az * bw - aw * bz
    n = jnp.maximum(jnp.sqrt(qw * qw + qx * qx + qy * qy + qz * qz),
                    jnp.float32(1e-12))
    qw_o[0] = qw / n
    qx_o[0] = qx / n
    qy_o[0] = qy / n
    qz_o[0] = qz / n


def _quat_prep(xa, ya, za, xb, yb, zb):
    spec = pl.BlockSpec((1, _NT, _P), lambda b: (b, 0, 0))
    shp = jax.ShapeDtypeStruct((_B, _NT, _P), jnp.float32)
    return pl.pallas_call(
        _k1_body,
        grid=(_B,),
        in_specs=[spec] * 6,
        out_specs=[spec] * 4,
        out_shape=[shp] * 4,
        compiler_params=pltpu.CompilerParams(
            dimension_semantics=("parallel",)),
    )(xa, ya, za, xb, yb, zb)


_HB = _RB // 2


def _k2_body(pmT_ref, pm_ref, qT_ref, q_ref, o5, o15, o40):
    pmT = pmT_ref[0]
    pm = pm_ref[0]
    qT = qT_ref[0]
    q = q_ref[0]
    g = jnp.dot(pm, pmT, preferred_element_type=jnp.float32)
    sqr = jnp.sum(pm * pm, axis=1, keepdims=True)
    sqc = jnp.sum(pmT * pmT, axis=0, keepdims=True)
    d = -((sqr + sqc) - 2.0 * g)
    dp = jnp.abs(jnp.dot(q, qT, preferred_element_type=jnp.float32,
                         precision=jax.lax.Precision.HIGHEST))
    amat = 2.0 * _acos_poly(jnp.clip(dp, 0.0, 1.0 - 1e-7))
    jidx = jax.lax.broadcasted_iota(jnp.int32, (_HB, _P), 1)
    neg = jnp.float32(-3.0e38)
    da, db = d[:_HB], d[_HB:]
    ama, amb = amat[:_HB], amat[_HB:]

    def extract1(dcur):
        m = jnp.max(dcur, axis=1, keepdims=True)
        cand = jnp.where(dcur >= m, jidx, jnp.int32(_P))
        amin = jnp.min(cand, axis=1, keepdims=True)
        return jnp.where(jidx == amin, neg, dcur)

    def step(_, carry):
        dca, dcb = carry
        return extract1(dca), extract1(dcb)

    def emit(out_ref, dca, dcb, k):
        sa = jnp.sum(jnp.where(dca == neg, ama, 0.0), axis=1)
        sb = jnp.sum(jnp.where(dcb == neg, amb, 0.0), axis=1)
        out_ref[...] = jnp.concatenate(
            [sa, sb]).reshape(1, 1, 1, _RB) * jnp.float32(1.0 / k)

    carry = (da, db)
    carry = jax.lax.fori_loop(0, _KS[0], step, carry)
    emit(o5, carry[0], carry[1], _KS[0])
    carry = jax.lax.fori_loop(_KS[0], _KS[1], step, carry)
    emit(o15, carry[0], carry[1], _KS[1])
    carry = jax.lax.fori_loop(_KS[1], _KS[2], step, carry)
    emit(o40, carry[0], carry[1], _KS[2])


def _knn_angles(pmT, pm, qT, q):
    full = pl.BlockSpec((1, 8, _P), lambda i, j: (i, 0, 0))
    rows = pl.BlockSpec((1, _RB, 8), lambda i, j: (i, j, 0))
    ospec = pl.BlockSpec((1, 1, 1, _RB), lambda i, j: (i, j, 0, 0))
    oshp = jax.ShapeDtypeStruct((_BT, _NRB, 1, _RB), jnp.float32)
    return pl.pallas_call(
        _k2_body,
        grid=(_BT, _NRB),
        in_specs=[full, rows, full, rows],
        out_specs=[ospec] * 3,
        out_shape=[oshp] * 3,
        compiler_params=pltpu.CompilerParams(
            dimension_semantics=("parallel", "parallel")),
    )(pmT, pm, qT, q)


def _kth_bits(bits, kk):
    def body(_, lohi):
        lo, hi = lohi
        mid = lo + ((hi - lo) >> 1)
        cnt = jnp.sum((bits <= mid).astype(jnp.int32))
        pred = cnt >= kk
        return (jnp.where(pred, lo, mid + 1), jnp.where(pred, mid, hi))

    lo, _ = jax.lax.fori_loop(
        0, 31, body, (jnp.int32(0), jnp.int32(0x7F7FFFFF)))
    return lo


def _k3_body(i5, i15, i40, r5, r15, r40):
    half = _B * _P // 2
    for inc_ref, out_ref in ((i5, r5), (i15, r15), (i40, r40)):
        mi = jnp.mean(inc_ref[...], axis=1)
        bits = jax.lax.bitcast_convert_type(mi, jnp.int32)
        v1 = jax.lax.bitcast_convert_type(_kth_bits(bits, half), jnp.float32)
        v2 = jax.lax.bitcast_convert_type(_kth_bits(bits, half + 1),
                                          jnp.float32)
        scale = jnp.maximum((v1 + v2) * 0.5, jnp.float32(1e-6))
        out_ref[...] = jnp.exp(-mi / scale).reshape(_B, 1, _P)


def _rigidity(inc5, inc15, inc40):
    ispec = pl.BlockSpec((_B, _NT, _P), lambda: (0, 0, 0))
    ospec = pl.BlockSpec((_B, 1, _P), lambda: (0, 0, 0))
    oshp = jax.ShapeDtypeStruct((_B, 1, _P), jnp.float32)
    return pl.pallas_call(
        _k3_body,
        grid=(),
        in_specs=[ispec] * 3,
        out_specs=[ospec] * 3,
        out_shape=[oshp] * 3,
    )(inc5, inc15, inc40)


def _k4_body(p0, p1, p2, p3, r5, r15, r40, w1t, b1c, w2t, b2c, mx_o, sm_o):
    f = pl.program_id(1)
    zero = jnp.zeros((1, _P), jnp.float32)
    xr = jnp.concatenate(
        [p0[0], p1[0], p2[0], p3[0], r5[0], r15[0], r40[0], zero], axis=0)
    h1 = jnp.dot(w1t[...], xr, preferred_element_type=jnp.float32) + b1c[...]
    h1 = _gelu(h1)
    h2 = jnp.dot(w2t[...], h1, preferred_element_type=jnp.float32) + b2c[...]
    h2 = _gelu(h2)
    mx = jnp.max(h2, axis=1).reshape(1, 1, -1)
    sm = jnp.sum(h2, axis=1).reshape(1, 1, -1)

    @pl.when(f == 0)
    def _init():
        mx_o[...] = mx
        sm_o[...] = sm

    @pl.when(f > 0)
    def _acc():
        mx_o[...] = jnp.maximum(mx_o[...], mx)
        sm_o[...] = sm_o[...] + sm


def _mlp_pool(pts, rigs, w1t, b1c, w2t, b2c):
    h1, h2 = w1t.shape[0], w2t.shape[0]
    pspec = pl.BlockSpec((1, 1, _P), lambda b, f: (b * _F + f, 0, 0))
    rspec = pl.BlockSpec((1, 1, _P), lambda b, f: (b, 0, 0))
    w1spec = pl.BlockSpec(w1t.shape, lambda b, f: (0, 0))
    b1spec = pl.BlockSpec(b1c.shape, lambda b, f: (0, 0))
    w2spec = pl.BlockSpec(w2t.shape, lambda b, f: (0, 0))
    b2spec = pl.BlockSpec(b2c.shape, lambda b, f: (0, 0))
    ospec = pl.BlockSpec((1, 1, h2), lambda b, f: (b, 0, 0))
    oshp = jax.ShapeDtypeStruct((_B, 1, h2), jnp.float32)
    return pl.pallas_call(
        _k4_body,
        grid=(_B, _F),
        in_specs=[pspec] * 4 + [rspec] * 3 + [w1spec, b1spec, w2spec, b2spec],
        out_specs=[ospec] * 2,
        out_shape=[oshp] * 2,
        compiler_params=pltpu.CompilerParams(
            dimension_semantics=("parallel", "arbitrary")),
    )(*pts, *rigs, w1t, b1c, w2t, b2c)


def _k5_body(mx, sm, wc, bc, out):
    pooled = jnp.concatenate(
        [mx[:, 0, :], sm[:, 0, :] * jnp.float32(1.0 / (_F * _P))], axis=1)
    out[...] = jnp.dot(pooled, wc[...],
                       preferred_element_type=jnp.float32) + bc[...]


def _head(mx, sm, wc, bcr):
    nc = wc.shape[1]
    return pl.pallas_call(
        _k5_body,
        grid=(),
        in_specs=[pl.BlockSpec(mx.shape, lambda: (0, 0, 0)),
                  pl.BlockSpec(sm.shape, lambda: (0, 0, 0)),
                  pl.BlockSpec(wc.shape, lambda: (0, 0)),
                  pl.BlockSpec(bcr.shape, lambda: (0, 0))],
        out_specs=pl.BlockSpec((_B, nc), lambda: (0, 0)),
        out_shape=jax.ShapeDtypeStruct((_B, nc), jnp.float32),
    )(mx, sm, wc, bcr)


def kernel(inputs, W1, b1, W2, b2, Wc, bc):
    f32 = jnp.float32
    x = inputs[..., 0].astype(f32)
    y = inputs[..., 1].astype(f32)
    z = inputs[..., 2].astype(f32)

    qw, qx, qy, qz = _quat_prep(x[:, :_NT], y[:, :_NT], z[:, :_NT],
                                x[:, 1:], y[:, 1:], z[:, 1:])

    zero = jnp.zeros((_B, _NT, _P), f32)
    pmT = jnp.stack([x[:, :_NT], y[:, :_NT], z[:, :_NT],
                     zero, zero, zero, zero, zero],
                    axis=2).reshape(_BT, 8, _P)
    qT = jnp.stack([qw, qx, qy, qz, zero, zero, zero, zero],
                   axis=2).reshape(_BT, 8, _P)
    pm = pmT.transpose(0, 2, 1)
    q = qT.transpose(0, 2, 1)

    inc5, inc15, inc40 = _knn_angles(pmT, pm, qT, q)
    inc5 = inc5.reshape(_B, _NT, _P)
    inc15 = inc15.reshape(_B, _NT, _P)
    inc40 = inc40.reshape(_B, _NT, _P)

    rigs = _rigidity(inc5, inc15, inc40)

    pts = [inputs[..., c].astype(f32).reshape(_B * _F, 1, _P)
           for c in range(4)]
    w1t = jnp.zeros((W1.shape[1], 8), f32).at[:, :7].set(W1.T.astype(f32))
    b1c = b1.astype(f32).reshape(-1, 1)
    w2t = W2.T.astype(f32)
    b2c = b2.astype(f32).reshape(-1, 1)
    mx, sm = _mlp_pool(pts, rigs, w1t, b1c, w2t, b2c)

    return _head(mx, sm, Wc.astype(f32), bc.astype(f32).reshape(1, -1))


# speedup vs baseline: 7.1668x; 1.1640x over previous
import functools

import jax
import jax.numpy as jnp
from jax.experimental import pallas as pl
from jax.experimental.pallas import tpu as pltpu

_B, _F, _P = 4, 8, 2048
_NT = _F - 1
_BT = _B * _NT
_RB = 512
_NRB = _P // _RB
_KS = (5, 15, 40)


def _acos_poly(x):
    p = jnp.float32(-0.0012624911)
    p = p * x + jnp.float32(0.0066700901)
    p = p * x + jnp.float32(-0.0170881256)
    p = p * x + jnp.float32(0.0308918810)
    p = p * x + jnp.float32(-0.0501743046)
    p = p * x + jnp.float32(0.0889789874)
    p = p * x + jnp.float32(-0.2145988016)
    p = p * x + jnp.float32(1.5707963050)
    return jnp.sqrt(jnp.maximum(1.0 - x, 0.0)) * p


def _erf_poly(x):
    ax = jnp.abs(x)
    t = 1.0 / (1.0 + jnp.float32(0.3275911) * ax)
    y = jnp.float32(1.061405429)
    y = y * t + jnp.float32(-1.453152027)
    y = y * t + jnp.float32(1.421413741)
    y = y * t + jnp.float32(-0.284496736)
    y = y * t + jnp.float32(0.254829592)
    y = 1.0 - y * t * jnp.exp(-ax * ax)
    return jnp.where(x >= 0, y, -y)


def _gelu(x):
    return x * 0.5 * (1.0 + _erf_poly(x * jnp.float32(0.7071067811865476)))


def _bearing(x, y, z):
    cx = (jnp.min(x, axis=1, keepdims=True) + jnp.max(x, axis=1, keepdims=True)) * 0.5
    cy = (jnp.min(y, axis=1, keepdims=True) + jnp.max(y, axis=1, keepdims=True)) * 0.5
    cz = (jnp.min(z, axis=1, keepdims=True) + jnp.max(z, axis=1, keepdims=True)) * 0.5
    dx, dy, dz = x - cx, y - cy, z - cz
    n = jnp.maximum(jnp.sqrt(dx * dx + dy * dy + dz * dz), jnp.float32(1e-12))
    dx, dy, dz = dx / n, dy / n, dz / n
    dot = jnp.clip(dy, -1.0 + 1e-7, 1.0 - 1e-7)
    w = jnp.sqrt((1.0 + dot) * 0.5)
    s = jnp.sqrt((1.0 - dot) * 0.5)
    an = jnp.maximum(jnp.sqrt(dx * dx + dz * dz), jnp.float32(1e-12))
    return w, dz / an * s, -dx / an * s


def _k1_body(xa, ya, za, xb, yb, zb, qw_o, qx_o, qy_o, qz_o):
    bw, bx, bz = _bearing(xa[0], ya[0], za[0])
    aw, ax, az = _bearing(xb[0], yb[0], zb[0])
    qw = aw * bw + ax * bx + az * bz
    qx = ax * bw - aw * bx
    qy = ax * bz - az * bx
    qz = az * bw - aw * bz
    n = jnp.maximum(jnp.sqrt(qw * qw + qx * qx + qy * qy + qz * qz),
                    jnp.float32(1e-12))
    qw_o[0] = qw / n
    qx_o[0] = qx / n
    qy_o[0] = qy / n
    qz_o[0] = qz / n


def _quat_prep(xa, ya, za, xb, yb, zb):
    spec = pl.BlockSpec((1, _NT, _P), lambda b: (b, 0, 0))
    shp = jax.ShapeDtypeStruct((_B, _NT, _P), jnp.float32)
    return pl.pallas_call(
        _k1_body,
        grid=(_B,),
        in_specs=[spec] * 6,
        out_specs=[spec] * 4,
        out_shape=[shp] * 4,
        compiler_params=pltpu.CompilerParams(
            dimension_semantics=("parallel",)),
    )(xa, ya, za, xb, yb, zb)


_HB = _RB // 2


def _k2_body(pmT_ref, pm_ref, qT_ref, q_ref, o5, o15, o40):
    pmT = pmT_ref[0]
    pm = pm_ref[0]
    qT = qT_ref[0]
    q = q_ref[0]
    g = jnp.dot(pm, pmT, preferred_element_type=jnp.float32)
    sqr = jnp.sum(pm * pm, axis=1, keepdims=True)
    sqc = jnp.sum(pmT * pmT, axis=0, keepdims=True)
    d = -((sqr + sqc) - 2.0 * g)
    dp = jnp.abs(jnp.dot(q, qT, preferred_element_type=jnp.float32,
                         precision=jax.lax.Precision.HIGHEST))
    amat = 2.0 * _acos_poly(jnp.clip(dp, 0.0, 1.0 - 1e-7))
    jidx = jax.lax.broadcasted_iota(jnp.int32, (_HB, _P), 1)
    neg = jnp.float32(-3.0e38)
    da, db = d[:_HB], d[_HB:]
    ama, amb = amat[:_HB], amat[_HB:]

    def extract1(dcur):
        m = jnp.max(dcur, axis=1, keepdims=True)
        cand = jnp.where(dcur >= m, jidx, jnp.int32(_P))
        amin = jnp.min(cand, axis=1, keepdims=True)
        return jnp.where(jidx == amin, neg, dcur)

    def step(_, carry):
        dca, dcb = carry
        return extract1(dca), extract1(dcb)

    def emit(out_ref, dca, dcb, k):
        sa = jnp.sum(jnp.where(dca == neg, ama, 0.0), axis=1)
        sb = jnp.sum(jnp.where(dcb == neg, amb, 0.0), axis=1)
        out_ref[...] = jnp.concatenate(
            [sa, sb]).reshape(1, 1, 1, _RB) * jnp.float32(1.0 / k)

    def rest_sum(dcur, amh, krem):
        b = jax.lax.bitcast_convert_type(dcur, jnp.int32)
        imin = jnp.int32(-2147483648)
        s = jnp.where(b >= 0, b, imin - b)

        def bis(_, lohi):
            lo, hi = lohi
            mid = (lo >> 1) + (hi >> 1) + 1
            cnt = jnp.sum((s >= mid).astype(jnp.int32), axis=1,
                          keepdims=True)
            take = cnt >= krem
            return (jnp.where(take, mid, lo), jnp.where(take, hi, mid - 1))

        lo0 = jnp.full((_HB, 1), imin, jnp.int32)
        hi0 = jnp.full((_HB, 1), 2147483647, jnp.int32)
        t, _ = jax.lax.fori_loop(0, 35, bis, (lo0, hi0))
        cstrict = jnp.sum((s > t).astype(jnp.int32), axis=1, keepdims=True)
        eq = (s == t).astype(jnp.int32)
        rank = eq
        for sh in (1, 2, 4, 8, 16, 32, 64, 128, 256, 512, 1024):
            rolled = pltpu.roll(rank, sh, 1)
            rank = rank + jnp.where(jidx >= sh, rolled, 0)
        tie = (eq > 0) & (rank <= (krem - cstrict))
        mask = (s > t) | tie
        return jnp.sum(jnp.where(mask, amh, 0.0), axis=1)

    carry = (da, db)
    carry = jax.lax.fori_loop(0, _KS[0], step, carry)
    emit(o5, carry[0], carry[1], _KS[0])
    carry = jax.lax.fori_loop(_KS[0], _KS[1], step, carry)
    emit(o15, carry[0], carry[1], _KS[1])
    dca, dcb = carry
    krem = _KS[2] - _KS[1]
    s15a = jnp.sum(jnp.where(dca == neg, ama, 0.0), axis=1)
    s15b = jnp.sum(jnp.where(dcb == neg, amb, 0.0), axis=1)
    sa = s15a + rest_sum(dca, ama, krem)
    sb = s15b + rest_sum(dcb, amb, krem)
    o40[...] = jnp.concatenate(
        [sa, sb]).reshape(1, 1, 1, _RB) * jnp.float32(1.0 / _KS[2])


def _knn_angles(pmT, pm, qT, q):
    full = pl.BlockSpec((1, 8, _P), lambda i, j: (i, 0, 0))
    rows = pl.BlockSpec((1, _RB, 8), lambda i, j: (i, j, 0))
    ospec = pl.BlockSpec((1, 1, 1, _RB), lambda i, j: (i, j, 0, 0))
    oshp = jax.ShapeDtypeStruct((_BT, _NRB, 1, _RB), jnp.float32)
    return pl.pallas_call(
        _k2_body,
        grid=(_BT, _NRB),
        in_specs=[full, rows, full, rows],
        out_specs=[ospec] * 3,
        out_shape=[oshp] * 3,
        compiler_params=pltpu.CompilerParams(
            dimension_semantics=("parallel", "parallel")),
    )(pmT, pm, qT, q)


def _kth_bits(bits, kk):
    def body(_, lohi):
        lo, hi = lohi
        mid = lo + ((hi - lo) >> 1)
        cnt = jnp.sum((bits <= mid).astype(jnp.int32))
        pred = cnt >= kk
        return (jnp.where(pred, lo, mid + 1), jnp.where(pred, mid, hi))

    lo, _ = jax.lax.fori_loop(
        0, 31, body, (jnp.int32(0), jnp.int32(0x7F7FFFFF)))
    return lo


def _k3_body(i5, i15, i40, r5, r15, r40):
    half = _B * _P // 2
    for inc_ref, out_ref in ((i5, r5), (i15, r15), (i40, r40)):
        mi = jnp.mean(inc_ref[...], axis=1)
        bits = jax.lax.bitcast_convert_type(mi, jnp.int32)
        v1 = jax.lax.bitcast_convert_type(_kth_bits(bits, half), jnp.float32)
        v2 = jax.lax.bitcast_convert_type(_kth_bits(bits, half + 1),
                                          jnp.float32)
        scale = jnp.maximum((v1 + v2) * 0.5, jnp.float32(1e-6))
        out_ref[...] = jnp.exp(-mi / scale).reshape(_B, 1, _P)


def _rigidity(inc5, inc15, inc40):
    ispec = pl.BlockSpec((_B, _NT, _P), lambda: (0, 0, 0))
    ospec = pl.BlockSpec((_B, 1, _P), lambda: (0, 0, 0))
    oshp = jax.ShapeDtypeStruct((_B, 1, _P), jnp.float32)
    return pl.pallas_call(
        _k3_body,
        grid=(),
        in_specs=[ispec] * 3,
        out_specs=[ospec] * 3,
        out_shape=[oshp] * 3,
    )(inc5, inc15, inc40)


def _k4_body(p0, p1, p2, p3, r5, r15, r40, w1t, b1c, w2t, b2c, mx_o, sm_o):
    f = pl.program_id(1)
    zero = jnp.zeros((1, _P), jnp.float32)
    xr = jnp.concatenate(
        [p0[0], p1[0], p2[0], p3[0], r5[0], r15[0], r40[0], zero], axis=0)
    h1 = jnp.dot(w1t[...], xr, preferred_element_type=jnp.float32) + b1c[...]
    h1 = _gelu(h1)
    h2 = jnp.dot(w2t[...], h1, preferred_element_type=jnp.float32) + b2c[...]
    h2 = _gelu(h2)
    mx = jnp.max(h2, axis=1).reshape(1, 1, -1)
    sm = jnp.sum(h2, axis=1).reshape(1, 1, -1)

    @pl.when(f == 0)
    def _init():
        mx_o[...] = mx
        sm_o[...] = sm

    @pl.when(f > 0)
    def _acc():
        mx_o[...] = jnp.maximum(mx_o[...], mx)
        sm_o[...] = sm_o[...] + sm


def _mlp_pool(pts, rigs, w1t, b1c, w2t, b2c):
    h1, h2 = w1t.shape[0], w2t.shape[0]
    pspec = pl.BlockSpec((1, 1, _P), lambda b, f: (b * _F + f, 0, 0))
    rspec = pl.BlockSpec((1, 1, _P), lambda b, f: (b, 0, 0))
    w1spec = pl.BlockSpec(w1t.shape, lambda b, f: (0, 0))
    b1spec = pl.BlockSpec(b1c.shape, lambda b, f: (0, 0))
    w2spec = pl.BlockSpec(w2t.shape, lambda b, f: (0, 0))
    b2spec = pl.BlockSpec(b2c.shape, lambda b, f: (0, 0))
    ospec = pl.BlockSpec((1, 1, h2), lambda b, f: (b, 0, 0))
    oshp = jax.ShapeDtypeStruct((_B, 1, h2), jnp.float32)
    return pl.pallas_call(
        _k4_body,
        grid=(_B, _F),
        in_specs=[pspec] * 4 + [rspec] * 3 + [w1spec, b1spec, w2spec, b2spec],
        out_specs=[ospec] * 2,
        out_shape=[oshp] * 2,
        compiler_params=pltpu.CompilerParams(
            dimension_semantics=("parallel", "arbitrary")),
    )(*pts, *rigs, w1t, b1c, w2t, b2c)


def _k5_body(mx, sm, wc, bc, out):
    pooled = jnp.concatenate(
        [mx[:, 0, :], sm[:, 0, :] * jnp.float32(1.0 / (_F * _P))], axis=1)
    out[...] = jnp.dot(pooled, wc[...],
                       preferred_element_type=jnp.float32) + bc[...]


def _head(mx, sm, wc, bcr):
    nc = wc.shape[1]
    return pl.pallas_call(
        _k5_body,
        grid=(),
        in_specs=[pl.BlockSpec(mx.shape, lambda: (0, 0, 0)),
                  pl.BlockSpec(sm.shape, lambda: (0, 0, 0)),
                  pl.BlockSpec(wc.shape, lambda: (0, 0)),
                  pl.BlockSpec(bcr.shape, lambda: (0, 0))],
        out_specs=pl.BlockSpec((_B, nc), lambda: (0, 0)),
        out_shape=jax.ShapeDtypeStruct((_B, nc), jnp.float32),
    )(mx, sm, wc, bcr)


def kernel(inputs, W1, b1, W2, b2, Wc, bc):
    f32 = jnp.float32
    x = inputs[..., 0].astype(f32)
    y = inputs[..., 1].astype(f32)
    z = inputs[..., 2].astype(f32)

    qw, qx, qy, qz = _quat_prep(x[:, :_NT], y[:, :_NT], z[:, :_NT],
                                x[:, 1:], y[:, 1:], z[:, 1:])

    zero = jnp.zeros((_B, _NT, _P), f32)
    pmT = jnp.stack([x[:, :_NT], y[:, :_NT], z[:, :_NT],
                     zero, zero, zero, zero, zero],
                    axis=2).reshape(_BT, 8, _P)
    qT = jnp.stack([qw, qx, qy, qz, zero, zero, zero, zero],
                   axis=2).reshape(_BT, 8, _P)
    pm = pmT.transpose(0, 2, 1)
    q = qT.transpose(0, 2, 1)

    inc5, inc15, inc40 = _knn_angles(pmT, pm, qT, q)
    inc5 = inc5.reshape(_B, _NT, _P)
    inc15 = inc15.reshape(_B, _NT, _P)
    inc40 = inc40.reshape(_B, _NT, _P)

    rigs = _rigidity(inc5, inc15, inc40)

    pts = [inputs[..., c].astype(f32).reshape(_B * _F, 1, _P)
           for c in range(4)]
    w1t = jnp.zeros((W1.shape[1], 8), f32).at[:, :7].set(W1.T.astype(f32))
    b1c = b1.astype(f32).reshape(-1, 1)
    w2t = W2.T.astype(f32)
    b2c = b2.astype(f32).reshape(-1, 1)
    mx, sm = _mlp_pool(pts, rigs, w1t, b1c, w2t, b2c)

    return _head(mx, sm, Wc.astype(f32), bc.astype(f32).reshape(1, -1))
